# baseline (speedup 1.0000x reference)
import functools

import jax
import jax.numpy as jnp
from jax import lax
from jax.experimental import pallas as pl
from jax.experimental.pallas import tpu as pltpu
from jax.experimental.pallas import tpu_sc as plsc

B, L = 4096, 50
N = B * L
D_W, D_P = 64, 16
D_WP = 128
D_OUT = D_W + D_P
NW = 32
BB = B // NW
CH_B = 8
CHUNK = CH_B * L
N_CH = BB // CH_B
NBUF = 3
PER_W = BB * L
SUB = CHUNK


def _make_kernel():
    mesh = plsc.VectorSubcoreMesh(core_axis_name="c", subcore_axis_name="s")

    @functools.partial(
        pl.kernel,
        mesh=mesh,
        compiler_params=pltpu.CompilerParams(use_tc_tiling_on_sc=False),
        out_type=jax.ShapeDtypeStruct((B, 56, 128), jnp.float32),
        scratch_types=[
            pltpu.VMEM((PER_W,), jnp.int32),
            pltpu.VMEM((PER_W,), jnp.int32),
            pltpu.VMEM((NBUF, CHUNK, D_W), jnp.float32),
            pltpu.VMEM((NBUF, CHUNK, D_P), jnp.float32),
        ] + [pltpu.SemaphoreType.DMA] * (2 * NBUF),
    )
    def k(idx_hbm, pidx_hbm, wtab_hbm, ptab_hbm, out_hbm,
          idx_v, pidx_v, wbuf, pbuf, *sems):
        wid = lax.axis_index("s") * 2 + lax.axis_index("c")
        base = wid * PER_W
        b_base = wid * BB
        pltpu.sync_copy(idx_hbm.at[pl.ds(base, PER_W)], idx_v)
        pltpu.sync_copy(pidx_hbm.at[pl.ds(base, PER_W)], pidx_v)

        sgs, sws = sems[:NBUF], sems[NBUF:]
        gathers = [None] * N_CH

        def start_gathers(c):
            s = c % NBUF
            off = c * CHUNK
            cps = []
            so = 0
            while so < CHUNK:
                sl = min(SUB, CHUNK - so)
                cps.append(pltpu.make_async_copy(
                    wtab_hbm.at[idx_v.at[pl.ds(off + so, sl)]],
                    wbuf.at[s, pl.ds(so, sl)], sgs[s]))
                cps.append(pltpu.make_async_copy(
                    ptab_hbm.at[pidx_v.at[pl.ds(off + so, sl)]],
                    pbuf.at[s, pl.ds(so, sl)], sgs[s]))
                so += sl
            for cp in cps:
                cp.start()
            gathers[c] = cps

        def write_copies(c):
            s = c % NBUF
            b0 = b_base + c * CH_B
            cps = []
            for r in range(CH_B):
                cps.append(pltpu.make_async_copy(
                    wbuf.at[s, pl.ds(r * L, L)],
                    out_hbm.at[b0 + r, pl.ds(0, L), pl.ds(0, D_W)], sws[s]))
                cps.append(pltpu.make_async_copy(
                    pbuf.at[s, pl.ds(r * L, L)],
                    out_hbm.at[b0 + r, pl.ds(0, L), pl.ds(D_W, D_P)], sws[s]))
            return cps

        def start_writes(c):
            for cp in gathers[c]:
                cp.wait()
            for cp in write_copies(c):
                cp.start()

        for c in range(NBUF):
            start_gathers(c)
        for c in range(N_CH):
            start_writes(c)
            if c + NBUF < N_CH:
                for cp in write_copies(c):
                    cp.wait()
                start_gathers(c + NBUF)
        for c in range(N_CH - NBUF, N_CH):
            for cp in write_copies(c):
                cp.wait()

    return k


_sc_lookup = _make_kernel()


def kernel(sents, masks, positions, word_table, pos_table):
    del masks
    idx2 = sents.reshape(N).astype(jnp.int32) * 2
    pidx = positions.reshape(N).astype(jnp.int32)
    wtab2m = jnp.pad(word_table, ((0, 0), (0, D_WP - D_W))).reshape(2 * 10**6, D_W)
    out6 = _sc_lookup(idx2, pidx, wtab2m, pos_table)
    return out6[:, :L, :D_OUT]
